# Initial kernel scaffold; baseline (speedup 1.0000x reference)
import jax
import jax.numpy as jnp
from jax.experimental import pallas as pl


def kernel(Data, Labels, W, perm):
    raise NotImplementedError("write your pallas kernel here")


# speedup vs baseline: 13.6440x; 13.6440x over previous
import functools

import jax
import jax.numpy as jnp
import numpy as np
from jax.experimental import pallas as pl
from jax.experimental.pallas import tpu as pltpu

_PERCENT = 0.1
_MAX_FINITE_HI = 0x7F7F


def _saliency_bits(w):
    rw = w.astype(jnp.bfloat16).astype(jnp.float32)
    bits = jax.lax.bitcast_convert_type(jnp.abs(rw), jnp.int32)
    return jax.lax.shift_right_logical(bits, 16)


def _threshold_body(w_ref, tb_ref, mc_ref, *, K, D):
    bits = _saliency_bits(w_ref[...])

    def bisect(_, lohi):
        lo, hi = lohi
        mid = lo + (hi - lo + 1) // 2
        cnt = jnp.sum((bits >= mid).astype(jnp.int32))
        take = cnt >= K
        return (jnp.where(take, mid, lo), jnp.where(take, hi, mid - 1))

    lo, _ = jax.lax.fori_loop(
        0, 16, bisect, (jnp.int32(0), jnp.int32(_MAX_FINITE_HI))
    )
    thresh = lo
    eq = bits == thresh
    n_greater = jnp.sum((bits > thresh).astype(jnp.int32))
    n_equal = jnp.sum(eq.astype(jnp.int32))
    need = K - n_greater

    tb_ref[0, 0, :] = jnp.full((128,), thresh, jnp.int32)
    mc_ref[0, 0, :] = jnp.full((128,), D, jnp.int32)

    @pl.when(n_equal != need)
    def _tie_break():
        idx = jax.lax.broadcasted_iota(jnp.int32, bits.shape, bits.ndim - 1)

        def bisect_idx(_, lohi):
            lo2, hi2 = lohi
            mid = (lo2 + hi2) // 2
            cm = jnp.sum((eq & (idx < mid)).astype(jnp.int32))
            take = cm >= need
            return (jnp.where(take, lo2, mid), jnp.where(take, mid, hi2))

        _, h = jax.lax.fori_loop(0, 18, bisect_idx, (jnp.int32(0), jnp.int32(D)))
        mc_ref[0, 0, :] = jnp.full((128,), h, jnp.int32)


def _select_body(lab_ref, prm_ref, flat_ref, shuf_ref, w_ref, tb_ref, mc_ref,
                 out_ref):
    del lab_ref, prm_ref
    tb = tb_ref[0, 0, 0]
    mc = mc_ref[0, 0, 0]
    bits = _saliency_bits(w_ref[...])
    idx = jax.lax.broadcasted_iota(jnp.int32, bits.shape, bits.ndim - 1)
    sel = (bits > tb) | ((bits == tb) & (idx < mc))
    out_ref[...] = jnp.where(sel, shuf_ref[...], flat_ref[...])


def kernel(Data, Labels, W, perm):
    B = Data.shape[0]
    D = int(np.prod(Data.shape[1:]))
    C = W.shape[1]
    K = int(np.floor(_PERCENT * D))

    flat = Data.reshape(B, 1, D)
    Wt = W.T.reshape(C, 1, D)

    tb, mc = pl.pallas_call(
        functools.partial(_threshold_body, K=K, D=D),
        grid=(C,),
        in_specs=[pl.BlockSpec((1, 1, D), lambda c: (c, 0, 0))],
        out_specs=[
            pl.BlockSpec((1, 1, 128), lambda c: (c, 0, 0)),
            pl.BlockSpec((1, 1, 128), lambda c: (c, 0, 0)),
        ],
        out_shape=[
            jax.ShapeDtypeStruct((C, 1, 128), jnp.int32),
            jax.ShapeDtypeStruct((C, 1, 128), jnp.int32),
        ],
    )(Wt)

    grid_spec = pltpu.PrefetchScalarGridSpec(
        num_scalar_prefetch=2,
        grid=(B,),
        in_specs=[
            pl.BlockSpec((1, 1, D), lambda i, lab, prm: (i, 0, 0)),
            pl.BlockSpec((1, 1, D), lambda i, lab, prm: (prm[i], 0, 0)),
            pl.BlockSpec((1, 1, D), lambda i, lab, prm: (lab[i], 0, 0)),
            pl.BlockSpec((1, 1, 128), lambda i, lab, prm: (lab[i], 0, 0)),
            pl.BlockSpec((1, 1, 128), lambda i, lab, prm: (lab[i], 0, 0)),
        ],
        out_specs=pl.BlockSpec((1, 1, D), lambda i, lab, prm: (i, 0, 0)),
    )
    new_flat = pl.pallas_call(
        _select_body,
        grid_spec=grid_spec,
        out_shape=jax.ShapeDtypeStruct((B, 1, D), jnp.float32),
    )(Labels, perm, flat, flat, Wt, tb, mc)

    return new_flat.reshape(Data.shape)


# speedup vs baseline: 34.4467x; 2.5247x over previous
import functools

import jax
import jax.numpy as jnp
import numpy as np
from jax.experimental import pallas as pl
from jax.experimental.pallas import tpu as pltpu

_PERCENT = 0.1
_MAX_FINITE_HI = 0x7F7F


def _pack_body(w_ref, sal_ref):
    w = w_ref[...]
    rw = jnp.abs(w.astype(jnp.bfloat16).astype(jnp.float32)).T
    sal_ref[...] = jax.lax.shift_right_logical(
        jax.lax.bitcast_convert_type(rw, jnp.int32), 16
    )


def _mask_body(sal_ref, mask_ref, *, K, D):
    bits = sal_ref[...]

    def bisect(_, lohi):
        lo, hi = lohi
        mid = lo + (hi - lo + 1) // 2
        cnt = jnp.sum((bits >= mid).astype(jnp.int32))
        take = cnt >= K
        return (jnp.where(take, mid, lo), jnp.where(take, hi, mid - 1))

    lo, _ = jax.lax.fori_loop(
        0, 15, bisect, (jnp.int32(0), jnp.int32(_MAX_FINITE_HI))
    )
    thresh = lo
    eq = bits == thresh
    n_greater = jnp.sum((bits > thresh).astype(jnp.int32))
    need = K - n_greater

    sub = jax.lax.broadcasted_iota(jnp.int32, bits.shape, 1)
    lane = jax.lax.broadcasted_iota(jnp.int32, bits.shape, 2)
    idx = sub * 128 + lane

    def bisect_idx(_, lohi):
        lo2, hi2 = lohi
        mid = (lo2 + hi2) // 2
        cm = jnp.sum((eq & (idx < mid)).astype(jnp.int32))
        take = cm >= need
        return (jnp.where(take, lo2, mid), jnp.where(take, mid, hi2))

    _, mcut = jax.lax.fori_loop(0, 18, bisect_idx, (jnp.int32(0), jnp.int32(D)))

    sel = (bits > thresh) | (eq & (idx < mcut))
    mask_ref[...] = sel.astype(jnp.float32)


def _select_body(lab_ref, prm_ref, flat_ref, shuf_ref, mask_ref, out_ref):
    del lab_ref, prm_ref
    out_ref[...] = jnp.where(mask_ref[...] != 0.0, shuf_ref[...], flat_ref[...])


def kernel(Data, Labels, W, perm):
    B = Data.shape[0]
    D = int(np.prod(Data.shape[1:]))
    C = W.shape[1]
    K = int(np.floor(_PERCENT * D))
    L = 128
    S = D // L
    R = 1024
    NT = D // R

    flat3 = Data.reshape(B, S, L)

    sal = pl.pallas_call(
        _pack_body,
        grid=(NT,),
        in_specs=[pl.BlockSpec((R, C), lambda t: (t, 0))],
        out_specs=pl.BlockSpec((C, R), lambda t: (0, t)),
        out_shape=jax.ShapeDtypeStruct((C, D), jnp.int32),
    )(W)
    sal3 = sal.reshape(C, S, L)

    mask = pl.pallas_call(
        functools.partial(_mask_body, K=K, D=D),
        grid=(C,),
        in_specs=[pl.BlockSpec((1, S, L), lambda c: (c, 0, 0))],
        out_specs=pl.BlockSpec((1, S, L), lambda c: (c, 0, 0)),
        out_shape=jax.ShapeDtypeStruct((C, S, L), jnp.float32),
    )(sal3)

    grid_spec = pltpu.PrefetchScalarGridSpec(
        num_scalar_prefetch=2,
        grid=(B,),
        in_specs=[
            pl.BlockSpec((1, S, L), lambda i, lab, prm: (i, 0, 0)),
            pl.BlockSpec((1, S, L), lambda i, lab, prm: (prm[i], 0, 0)),
            pl.BlockSpec((1, S, L), lambda i, lab, prm: (lab[i], 0, 0)),
        ],
        out_specs=pl.BlockSpec((1, S, L), lambda i, lab, prm: (i, 0, 0)),
    )
    new_flat = pl.pallas_call(
        _select_body,
        grid_spec=grid_spec,
        out_shape=jax.ShapeDtypeStruct((B, S, L), jnp.float32),
    )(Labels, perm, flat3, flat3, mask)

    return new_flat.reshape(Data.shape)


# speedup vs baseline: 94.5943x; 2.7461x over previous
import functools

import jax
import jax.numpy as jnp
import numpy as np
from jax.experimental import pallas as pl
from jax.experimental.pallas import tpu as pltpu

_PERCENT = 0.1
_MAX_FINITE_HI = 0x7F7F
_CT = 8


def _mask_body(w_ref, mask_ref, sal_ref, *, K, D):
    w = w_ref[...]
    rw = jnp.abs(w.astype(jnp.bfloat16).astype(jnp.float32))
    sal_ref[...] = jax.lax.shift_right_logical(
        jax.lax.bitcast_convert_type(rw, jnp.int32), 16
    )

    def bisect(_, lohi):
        lo, hi = lohi
        mid = lo + (hi - lo + 1) // 2
        cnt = jnp.sum((sal_ref[...] >= mid).astype(jnp.int32), axis=1,
                      keepdims=True)
        take = cnt >= K
        return (jnp.where(take, mid, lo), jnp.where(take, hi, mid - 1))

    thresh, _ = jax.lax.fori_loop(
        0, 15, bisect,
        (jnp.zeros((_CT, 1), jnp.int32),
         jnp.full((_CT, 1), _MAX_FINITE_HI, jnp.int32)),
    )

    bits = sal_ref[...]
    eq = bits == thresh
    n_greater = jnp.sum((bits > thresh).astype(jnp.int32), axis=1,
                        keepdims=True)
    need = K - n_greater
    idx = jax.lax.broadcasted_iota(jnp.int32, bits.shape, 1)

    def bisect_idx(_, lohi):
        lo2, hi2 = lohi
        mid = (lo2 + hi2) // 2
        cm = jnp.sum((eq & (idx < mid)).astype(jnp.int32), axis=1,
                     keepdims=True)
        take = cm >= need
        return (jnp.where(take, lo2, mid), jnp.where(take, mid, hi2))

    _, mcut = jax.lax.fori_loop(
        0, 18, bisect_idx,
        (jnp.zeros((_CT, 1), jnp.int32), jnp.full((_CT, 1), D, jnp.int32)),
    )

    sel = (bits > thresh) | (eq & (idx < mcut))
    mask_ref[...] = sel.astype(jnp.float32)


def _split3(x):
    hi = x.astype(jnp.bfloat16)
    r1 = x - hi.astype(jnp.float32)
    mid = r1.astype(jnp.bfloat16)
    lo = (r1 - mid.astype(jnp.float32)).astype(jnp.bfloat16)
    return hi, mid, lo


def _select_body(x_ref, m_ref, p_ref, oh_ref, out_ref):
    x = x_ref[...]
    pb = p_ref[...]
    dn = (((1,), (0,)), ((), ()))
    xh, xm, xl = _split3(x)
    f32 = jnp.float32
    xs = (jax.lax.dot_general(xh, pb, dn, preferred_element_type=f32)
          + jax.lax.dot_general(xm, pb, dn, preferred_element_type=f32)
          + jax.lax.dot_general(xl, pb, dn, preferred_element_type=f32))
    mb = jax.lax.dot_general(m_ref[...], oh_ref[...], (((0,), (0,)), ((), ())),
                             preferred_element_type=f32)
    out_ref[...] = jnp.where(mb != 0.0, xs, x)


def kernel(Data, Labels, W, perm):
    B = Data.shape[0]
    H, Wd, Ch = Data.shape[1], Data.shape[2], Data.shape[3]
    D = H * Wd * Ch
    C = W.shape[1]
    K = int(np.floor(_PERCENT * D))
    CH = 3072 if D % 3072 == 0 else D
    NC = D // CH

    X = Data.transpose(1, 3, 2, 0).reshape(D, B)
    Wt = W.T

    mask = pl.pallas_call(
        functools.partial(_mask_body, K=K, D=D),
        grid=((C + _CT - 1) // _CT,),
        in_specs=[pl.BlockSpec((_CT, D), lambda c: (c, 0))],
        out_specs=pl.BlockSpec((_CT, D), lambda c: (c, 0)),
        out_shape=jax.ShapeDtypeStruct((C, D), jnp.float32),
        scratch_shapes=[pltpu.VMEM((_CT, D), jnp.int32)],
    )(Wt)

    maskP = (mask.reshape(C, H, Wd, Ch).transpose(0, 1, 3, 2)
             .reshape(C, D).astype(jnp.bfloat16))

    ar = jnp.arange(B, dtype=jnp.int32)
    P = (ar[:, None] == perm[None, :]).astype(jnp.bfloat16)
    oh = (jnp.arange(C, dtype=jnp.int32)[:, None] == Labels[None, :]
          ).astype(jnp.bfloat16)

    outX = pl.pallas_call(
        _select_body,
        grid=(NC,),
        in_specs=[
            pl.BlockSpec((CH, B), lambda t: (t, 0)),
            pl.BlockSpec((C, CH), lambda t: (0, t)),
            pl.BlockSpec((B, B), lambda t: (0, 0)),
            pl.BlockSpec((C, B), lambda t: (0, 0)),
        ],
        out_specs=pl.BlockSpec((CH, B), lambda t: (t, 0)),
        out_shape=jax.ShapeDtypeStruct((D, B), jnp.float32),
    )(X, maskP, P, oh)

    return outX.reshape(H, Ch, Wd, B).transpose(3, 0, 2, 1)
